# Initial kernel scaffold; baseline (speedup 1.0000x reference)
import functools

import jax
import jax.numpy as jnp
from jax import lax
from jax.experimental import pallas as pl
from jax.experimental.pallas import tpu as pltpu
from jax.experimental.pallas import tpu_sc as plsc

MAX_LEN = 8192
EMB = 64

NC = 2
NS = 16
NW = NC * NS

CHUNK = 128
NBUF = 8


def _fuse_body(w_ref, p_ref, o_ref):
    o_ref[...] = w_ref[...] + p_ref[...]


def _fuse_tables(word_head, pos_table):
    return pl.pallas_call(
        _fuse_body,
        out_shape=jax.ShapeDtypeStruct((MAX_LEN, EMB), jnp.float32),
    )(word_head, pos_table)


def _gather_kernel(n_tokens):
    assert n_tokens % (NW * CHUNK) == 0
    per_w = n_tokens // NW
    n_chunks = per_w // CHUNK
    assert n_chunks % NBUF == 0
    n_groups = n_chunks // NBUF

    mesh = plsc.VectorSubcoreMesh(core_axis_name="c", subcore_axis_name="s")

    @functools.partial(
        pl.kernel,
        out_type=jax.ShapeDtypeStruct((n_tokens, EMB), jnp.float32),
        mesh=mesh,
        scratch_types=[
            pltpu.VMEM((n_chunks, CHUNK), jnp.int32),
            pltpu.VMEM((NBUF, CHUNK, EMB), jnp.float32),
            pltpu.SemaphoreType.DMA,
            pltpu.SemaphoreType.DMA,
        ],
    )
    def k(idx_hbm, table_hbm, out_hbm, idx_v, rows_v, gsem, osem):
        wid = lax.axis_index("s") * NC + lax.axis_index("c")
        base = wid * per_w
        pltpu.sync_copy(idx_hbm.at[wid], idx_v)

        def group(g, _):
            c0 = g * NBUF
            for b in range(NBUF):
                pltpu.async_copy(table_hbm.at[idx_v.at[c0 + b]],
                                 rows_v.at[b], gsem)
            for b in range(NBUF):
                pltpu.make_async_copy(table_hbm.at[idx_v.at[c0 + b]],
                                      rows_v.at[b], gsem).wait()
                pltpu.async_copy(
                    rows_v.at[b],
                    out_hbm.at[pl.ds(base + (c0 + b) * CHUNK, CHUNK)],
                    osem)
            for b in range(NBUF):
                pltpu.make_async_copy(
                    rows_v.at[b],
                    out_hbm.at[pl.ds(base + (c0 + b) * CHUNK, CHUNK)],
                    osem).wait()
            return 0

        lax.fori_loop(0, n_groups, group, 0)

    return k


def kernel(X, word_table, pos_table):
    B, T = X.shape
    n_tokens = B * T
    fused = _fuse_tables(word_table[:MAX_LEN], pos_table)
    idx = X.reshape(NW, n_tokens // (NW * CHUNK), CHUNK)
    out = _gather_kernel(n_tokens)(idx, fused)
    return out.reshape(B, T, EMB)


# speedup vs baseline: 8.8971x; 8.8971x over previous
import functools

import jax
import jax.numpy as jnp
from jax import lax
from jax.experimental import pallas as pl
from jax.experimental.pallas import tpu as pltpu
from jax.experimental.pallas import tpu_sc as plsc

MAX_LEN = 8192
EMB = 64

NC = 2
NS = 16
NW = NC * NS

CHUNK = 128
NBUF = 8


def _fuse_body(w_ref, p_ref, o_ref):
    o_ref[...] = w_ref[...] + p_ref[...]


def _fuse_tables(word_head, pos_table):
    return pl.pallas_call(
        _fuse_body,
        out_shape=jax.ShapeDtypeStruct((MAX_LEN, EMB), jnp.float32),
    )(word_head, pos_table)


def _gather_kernel(n_tokens):
    assert n_tokens % (NW * CHUNK) == 0
    per_w = n_tokens // NW
    n_chunks = per_w // CHUNK
    assert n_chunks % NBUF == 0
    n_groups = n_chunks // NBUF

    mesh = plsc.VectorSubcoreMesh(core_axis_name="c", subcore_axis_name="s")

    @functools.partial(
        pl.kernel,
        out_type=jax.ShapeDtypeStruct((n_tokens, EMB), jnp.float32),
        mesh=mesh,
        scratch_types=[
            pltpu.VMEM((n_chunks, CHUNK), jnp.int32),
            pltpu.VMEM((NBUF, CHUNK, EMB), jnp.float32),
            pltpu.SemaphoreType.DMA,
            pltpu.SemaphoreType.DMA,
        ],
        compiler_params=pltpu.CompilerParams(use_tc_tiling_on_sc=False),
    )
    def k(idx_hbm, table_hbm, out_hbm, idx_v, rows_v, gsem, osem):
        wid = lax.axis_index("s") * NC + lax.axis_index("c")
        base = wid * per_w
        pltpu.sync_copy(idx_hbm.at[wid], idx_v)

        def group(g, _):
            c0 = g * NBUF
            for b in range(NBUF):
                pltpu.async_copy(table_hbm.at[idx_v.at[c0 + b]],
                                 rows_v.at[b], gsem)
            for b in range(NBUF):
                pltpu.make_async_copy(table_hbm.at[idx_v.at[c0 + b]],
                                      rows_v.at[b], gsem).wait()
                pltpu.async_copy(
                    rows_v.at[b],
                    out_hbm.at[pl.ds(base + (c0 + b) * CHUNK, CHUNK)],
                    osem)
            for b in range(NBUF):
                pltpu.make_async_copy(
                    rows_v.at[b],
                    out_hbm.at[pl.ds(base + (c0 + b) * CHUNK, CHUNK)],
                    osem).wait()
            return 0

        lax.fori_loop(0, n_groups, group, 0)

    return k


def kernel(X, word_table, pos_table):
    B, T = X.shape
    n_tokens = B * T
    fused = _fuse_tables(word_table[:MAX_LEN], pos_table)
    idx = X.reshape(NW, n_tokens // (NW * CHUNK), CHUNK)
    out = _gather_kernel(n_tokens)(idx, fused)
    return out.reshape(B, T, EMB)
